# baseline (speedup 1.0000x reference)
import jax
import jax.numpy as jnp
from jax import lax
from jax.experimental import pallas as pl
from jax.experimental.pallas import tpu as pltpu
from jax.experimental.pallas import tpu_sc as plsc

N = 10000
D = 128
HD = 64
C = 8
E = 320000
B = 250
NPG = 40
APG = 20
H = 64

CHUNK = 128
NSUB = 16
E_PAD = 327680
NCHUNK = E_PAD // CHUNK
PER_W = NCHUNK // NSUB
N_ACC = 10112
ROWS_PER_SUB = N_ACC // 16
OUT_STRIPE = 624
OUT_REM = N - 16 * OUT_STRIPE

GB = 25
NSTEP = B // GB


def _sc_edge_body(znf_hbm, src_hbm, dst_hbm, zero_hbm, out_hbm,
                  src_v, dst_v, r0, r1, r2, r3, agg_sh,
                  g0, g1, g2, g3, s0, s1, s2, s3):
    rows = (r0, r1, r2, r3)
    gsem = (g0, g1, g2, g3)
    ssem = (s0, s1, s2, s3)
    cid = lax.axis_index("c")
    sid = lax.axis_index("s")
    tbl = znf_hbm.at[cid]

    pltpu.sync_copy(src_hbm.at[sid], src_v)
    pltpu.sync_copy(dst_hbm.at[sid], dst_v)

    if True:
        pltpu.sync_copy(zero_hbm,
                        agg_sh.at[pl.ds(sid * ROWS_PER_SUB, ROWS_PER_SUB)])
        plsc.subcore_barrier()

        for b in range(2):
            pltpu.async_copy(tbl.at[src_v.at[b]], rows[b], gsem[b])

        @pl.loop(0, PER_W, step=4)
        def _(j0):
            for b in range(4):
                j = j0 + b
                b2 = (b + 2) % 4
                pltpu.make_async_copy(tbl.at[src_v.at[0]], rows[b],
                                      gsem[b]).wait()
                pltpu.async_copy(rows[b], agg_sh.at[dst_v.at[j]],
                                 ssem[b], add=True)

                @pl.when(j + 2 < PER_W)
                def _():
                    @pl.when(j >= 2)
                    def _():
                        pltpu.make_async_copy(tbl.at[src_v.at[0]],
                                              rows[b2], ssem[b2]).wait()
                    pltpu.async_copy(tbl.at[src_v.at[j + 2]], rows[b2],
                                     gsem[b2])

        for b in range(4):
            pltpu.make_async_copy(tbl.at[src_v.at[0]], rows[b],
                                  ssem[b]).wait()

        plsc.subcore_barrier()
        pltpu.sync_copy(agg_sh.at[pl.ds(sid * OUT_STRIPE, OUT_STRIPE)],
                        out_hbm.at[cid].at[pl.ds(sid * OUT_STRIPE,
                                                 OUT_STRIPE)])

        @pl.when(sid == 15)
        def _():
            pltpu.sync_copy(agg_sh.at[pl.ds(16 * OUT_STRIPE, OUT_REM)],
                            out_hbm.at[cid].at[pl.ds(16 * OUT_STRIPE,
                                                     OUT_REM)])


def _sc_edge_agg(znf, src3, dst3, zeros_acc):
    mesh = plsc.VectorSubcoreMesh(core_axis_name="c", subcore_axis_name="s")
    kfn = pl.kernel(
        _sc_edge_body,
        out_type=jax.ShapeDtypeStruct((2, N, HD), jnp.float32),
        mesh=mesh,
        scratch_types=[
            pltpu.VMEM((PER_W, CHUNK), jnp.int32),
            pltpu.VMEM((PER_W, CHUNK), jnp.int32),
        ] + [pltpu.VMEM((CHUNK, HD), jnp.float32)] * 4 + [
            pltpu.VMEM_SHARED((N_ACC, HD), jnp.float32),
        ] + [pltpu.SemaphoreType.DMA] * 8,
        compiler_params=pltpu.CompilerParams(use_tc_tiling_on_sc=False),
    )
    return kfn(znf, src3, dst3, zeros_acc)


def _tc_post_body(nf_ref, agg_ref, qs_ref, wself_ref, wnlo_ref, wnhi_ref,
                  bw_ref, w1_ref, b1_ref, w2_ref, b2_ref,
                  qout_ref, ws_ref, wf_ref, an_ref, normed_ref):
    nf3 = nf_ref[...]
    nf2 = nf3.reshape(GB * NPG, D)
    aggp = agg_ref[...]
    agglo = aggp[0].reshape(GB * NPG, HD)
    agghi = aggp[1].reshape(GB * NPG, HD)

    zs = (jnp.dot(nf2, wself_ref[...], preferred_element_type=jnp.float32)
          + jnp.dot(agglo, wnlo_ref[...], preferred_element_type=jnp.float32)
          + jnp.dot(agghi, wnhi_ref[...], preferred_element_type=jnp.float32)
          + bw_ref[...][0][None, :])
    zs = jnp.maximum(zs, 0.0)
    z3 = zs.reshape(GB, NPG, C)[:, :APG, :]
    z3 = jnp.clip(z3, 1e-10, 10.0)
    z3 = z3 - jnp.max(z3, axis=-1, keepdims=True)
    ez = jnp.exp(z3)
    ws3 = ez / jnp.sum(ez, axis=-1, keepdims=True)
    ws_ref[...] = ws3

    anf = nf3[:, :APG, :]
    wf = jax.lax.dot_general(ws3, anf, (((1,), (1,)), ((0,), (0,))),
                             preferred_element_type=jnp.float32,
                             precision=jax.lax.Precision.HIGHEST)
    wf_ref[...] = wf

    gdp = jax.lax.dot_general(anf, wf, (((2,), (2,)), ((0,), (0,))),
                              preferred_element_type=jnp.float32,
                              precision=jax.lax.Precision.HIGHEST)
    nfn = jnp.sqrt(jnp.sum(anf * anf, axis=2))
    wfn = jnp.sqrt(jnp.sum(wf * wf, axis=2))
    an = gdp / (nfn[:, :, None] * wfn[:, None, :])
    an_ref[...] = an
    normed_ref[...] = jnp.concatenate(
        [an, jnp.zeros((GB, NPG - APG, C), jnp.float32)], axis=1)

    q_agg = jnp.sum(qs_ref[...] * ws3, axis=1)
    snf = jnp.sum(nf3, axis=1)
    h = jnp.maximum(
        jnp.dot(snf, w1_ref[...], preferred_element_type=jnp.float32)
        + b1_ref[...][0][None, :], 0.0)
    qv = (jnp.dot(h, w2_ref[...], preferred_element_type=jnp.float32)
          + b2_ref[...][0][None, :])
    qout_ref[...] = (q_agg + qv)[:, None, :]


def _tc_post(nf3, agg4, qs2, W_self, Wn_lo, Wn_hi, b_w, W1, b1, W2, b2):
    grid = (NSTEP,)
    out_shapes = (
        jax.ShapeDtypeStruct((B, 1, C), jnp.float32),
        jax.ShapeDtypeStruct((B, APG, C), jnp.float32),
        jax.ShapeDtypeStruct((B, C, D), jnp.float32),
        jax.ShapeDtypeStruct((B, APG, C), jnp.float32),
        jax.ShapeDtypeStruct((B, NPG, C), jnp.float32),
    )
    in_specs = [
        pl.BlockSpec((GB, NPG, D), lambda i: (i, 0, 0)),
        pl.BlockSpec((2, GB, NPG, HD), lambda i: (0, i, 0, 0)),
        pl.BlockSpec((GB, APG, 1), lambda i: (i, 0, 0)),
        pl.BlockSpec((D, C), lambda i: (0, 0)),
        pl.BlockSpec((HD, C), lambda i: (0, 0)),
        pl.BlockSpec((HD, C), lambda i: (0, 0)),
        pl.BlockSpec((1, C), lambda i: (0, 0)),
        pl.BlockSpec((D, H), lambda i: (0, 0)),
        pl.BlockSpec((1, H), lambda i: (0, 0)),
        pl.BlockSpec((H, 1), lambda i: (0, 0)),
        pl.BlockSpec((1, 1), lambda i: (0, 0)),
    ]
    out_specs = (
        pl.BlockSpec((GB, 1, C), lambda i: (i, 0, 0)),
        pl.BlockSpec((GB, APG, C), lambda i: (i, 0, 0)),
        pl.BlockSpec((GB, C, D), lambda i: (i, 0, 0)),
        pl.BlockSpec((GB, APG, C), lambda i: (i, 0, 0)),
        pl.BlockSpec((GB, NPG, C), lambda i: (i, 0, 0)),
    )
    return pl.pallas_call(
        _tc_post_body, grid=grid, in_specs=in_specs, out_specs=out_specs,
        out_shape=out_shapes,
    )(nf3, agg4, qs2, W_self, Wn_lo, Wn_hi, b_w, W1, b1, W2, b2)


def kernel(node_feature, qs, edge_index, graph_id, ally_indices,
           W_self, W_nbr, b_w, W1, b1, W2, b2):
    del graph_id, ally_indices

    pad = E_PAD - E
    src3 = jnp.concatenate([edge_index[0], jnp.zeros((pad,), jnp.int32)]
                           ).reshape(NSUB, PER_W, CHUNK)
    dst3 = jnp.concatenate([edge_index[1], jnp.full((pad,), N, jnp.int32)]
                           ).reshape(NSUB, PER_W, CHUNK)
    znf = jnp.stack([node_feature[:, :HD], node_feature[:, HD:]])
    zeros_acc = jnp.zeros((ROWS_PER_SUB, HD), jnp.float32)
    agg_halves = _sc_edge_agg(znf, src3, dst3, zeros_acc)

    nf3 = node_feature.reshape(B, NPG, D)
    agg4 = agg_halves.reshape(2, B, NPG, HD)
    qs2 = qs.reshape(B, APG, 1)
    q_out, ws3, wf, an3, normed3 = _tc_post(
        nf3, agg4, qs2, W_self, W_nbr[:HD], W_nbr[HD:], b_w.reshape(1, C),
        W1, b1.reshape(1, H), W2, b2.reshape(1, 1))

    return (q_out.reshape(B, C), ws3.reshape(B * APG, C), wf,
            an3.reshape(B * APG, C), normed3.reshape(N, C))


# speedup vs baseline: 1.0217x; 1.0217x over previous
import jax
import jax.numpy as jnp
from jax import lax
from jax.experimental import pallas as pl
from jax.experimental.pallas import tpu as pltpu
from jax.experimental.pallas import tpu_sc as plsc

N = 10000
D = 128
HD = 64
C = 8
E = 320000
B = 250
NPG = 40
APG = 20
H = 64

CHUNK = 128
NSUB = 16
E_PAD = 327680
NCHUNK = E_PAD // CHUNK
PER_W = NCHUNK // NSUB
N_ACC = 10112
ROWS_PER_SUB = N_ACC // 16
OUT_STRIPE = 624
OUT_REM = N - 16 * OUT_STRIPE

GB = 50
NSTEP = B // GB


def _sc_edge_body(znf_hbm, src_hbm, dst_hbm, zero_hbm, out_hbm,
                  src_v, dst_v, r0, r1, r2, r3, agg_sh,
                  g0, g1, g2, g3):
    rows = (r0, r1, r2, r3)
    gsem = (g0, g1, g2, g3)
    cid = lax.axis_index("c")
    sid = lax.axis_index("s")
    tbl = znf_hbm.at[cid]

    pltpu.sync_copy(zero_hbm,
                    agg_sh.at[pl.ds(sid * ROWS_PER_SUB, ROWS_PER_SUB)])
    pltpu.sync_copy(src_hbm.at[sid], src_v)
    pltpu.sync_copy(dst_hbm.at[sid], dst_v)
    plsc.subcore_barrier()

    for b in range(3):
        pltpu.async_copy(tbl.at[src_v.at[b]], rows[b], gsem[b])

    @pl.loop(0, PER_W, step=4)
    def _(j0):
        for b in range(4):
            j = j0 + b
            b2 = (b + 3) % 4
            pltpu.make_async_copy(tbl.at[src_v.at[0]], rows[b],
                                  gsem[b]).wait()

            @pl.when(j + 3 < PER_W)
            def _():
                pltpu.async_copy(tbl.at[src_v.at[j + 3]], rows[b2],
                                 gsem[b2])

            pltpu.sync_copy(rows[b], agg_sh.at[dst_v.at[j]],
                            add=True)

    plsc.subcore_barrier()
    pltpu.sync_copy(agg_sh.at[pl.ds(sid * OUT_STRIPE, OUT_STRIPE)],
                    out_hbm.at[cid].at[pl.ds(sid * OUT_STRIPE, OUT_STRIPE)])

    @pl.when(sid == 15)
    def _():
        pltpu.sync_copy(agg_sh.at[pl.ds(16 * OUT_STRIPE, OUT_REM)],
                        out_hbm.at[cid].at[pl.ds(16 * OUT_STRIPE, OUT_REM)])


def _sc_edge_agg(znf, src3, dst3, zeros_acc):
    mesh = plsc.VectorSubcoreMesh(core_axis_name="c", subcore_axis_name="s")
    kfn = pl.kernel(
        _sc_edge_body,
        out_type=jax.ShapeDtypeStruct((2, N, HD), jnp.float32),
        mesh=mesh,
        scratch_types=[
            pltpu.VMEM((PER_W, CHUNK), jnp.int32),
            pltpu.VMEM((PER_W, CHUNK), jnp.int32),
        ] + [pltpu.VMEM((CHUNK, HD), jnp.float32)] * 4 + [
            pltpu.VMEM_SHARED((N_ACC, HD), jnp.float32),
        ] + [pltpu.SemaphoreType.DMA] * 4,
        compiler_params=pltpu.CompilerParams(use_tc_tiling_on_sc=False),
    )
    return kfn(znf, src3, dst3, zeros_acc)


def _tc_post_body(nf_ref, agg_ref, qs_ref, wself_ref, wnlo_ref, wnhi_ref,
                  bw_ref, w1_ref, b1_ref, w2_ref, b2_ref,
                  qout_ref, ws_ref, wf_ref, an_ref, normed_ref):
    nf3 = nf_ref[...]
    nf2 = nf3.reshape(GB * NPG, D)
    aggp = agg_ref[...]
    agglo = aggp[0].reshape(GB * NPG, HD)
    agghi = aggp[1].reshape(GB * NPG, HD)

    zs = (jnp.dot(nf2, wself_ref[...], preferred_element_type=jnp.float32)
          + jnp.dot(agglo, wnlo_ref[...], preferred_element_type=jnp.float32)
          + jnp.dot(agghi, wnhi_ref[...], preferred_element_type=jnp.float32)
          + bw_ref[...][0][None, :])
    zs = jnp.maximum(zs, 0.0)
    z3 = zs.reshape(GB, NPG, C)[:, :APG, :]
    z3 = jnp.clip(z3, 1e-10, 10.0)
    z3 = z3 - jnp.max(z3, axis=-1, keepdims=True)
    ez = jnp.exp(z3)
    ws3 = ez / jnp.sum(ez, axis=-1, keepdims=True)
    ws_ref[...] = ws3

    anf = nf3[:, :APG, :]
    wf = jax.lax.dot_general(ws3, anf, (((1,), (1,)), ((0,), (0,))),
                             preferred_element_type=jnp.float32,
                             precision=jax.lax.Precision.HIGHEST)
    wf_ref[...] = wf

    gdp = jax.lax.dot_general(anf, wf, (((2,), (2,)), ((0,), (0,))),
                              preferred_element_type=jnp.float32,
                              precision=jax.lax.Precision.HIGHEST)
    nfn = jnp.sqrt(jnp.sum(anf * anf, axis=2))
    wfn = jnp.sqrt(jnp.sum(wf * wf, axis=2))
    an = gdp / (nfn[:, :, None] * wfn[:, None, :])
    an_ref[...] = an
    normed_ref[...] = jnp.concatenate(
        [an, jnp.zeros((GB, NPG - APG, C), jnp.float32)], axis=1)

    q_agg = jnp.sum(qs_ref[...] * ws3, axis=1)
    snf = jnp.sum(nf3, axis=1)
    h = jnp.maximum(
        jnp.dot(snf, w1_ref[...], preferred_element_type=jnp.float32)
        + b1_ref[...][0][None, :], 0.0)
    qv = (jnp.dot(h, w2_ref[...], preferred_element_type=jnp.float32)
          + b2_ref[...][0][None, :])
    qout_ref[...] = (q_agg + qv)[:, None, :]


def _tc_post(nf3, agg4, qs2, W_self, Wn_lo, Wn_hi, b_w, W1, b1, W2, b2):
    grid = (NSTEP,)
    out_shapes = (
        jax.ShapeDtypeStruct((B, 1, C), jnp.float32),
        jax.ShapeDtypeStruct((B, APG, C), jnp.float32),
        jax.ShapeDtypeStruct((B, C, D), jnp.float32),
        jax.ShapeDtypeStruct((B, APG, C), jnp.float32),
        jax.ShapeDtypeStruct((B, NPG, C), jnp.float32),
    )
    in_specs = [
        pl.BlockSpec((GB, NPG, D), lambda i: (i, 0, 0)),
        pl.BlockSpec((2, GB, NPG, HD), lambda i: (0, i, 0, 0)),
        pl.BlockSpec((GB, APG, 1), lambda i: (i, 0, 0)),
        pl.BlockSpec((D, C), lambda i: (0, 0)),
        pl.BlockSpec((HD, C), lambda i: (0, 0)),
        pl.BlockSpec((HD, C), lambda i: (0, 0)),
        pl.BlockSpec((1, C), lambda i: (0, 0)),
        pl.BlockSpec((D, H), lambda i: (0, 0)),
        pl.BlockSpec((1, H), lambda i: (0, 0)),
        pl.BlockSpec((H, 1), lambda i: (0, 0)),
        pl.BlockSpec((1, 1), lambda i: (0, 0)),
    ]
    out_specs = (
        pl.BlockSpec((GB, 1, C), lambda i: (i, 0, 0)),
        pl.BlockSpec((GB, APG, C), lambda i: (i, 0, 0)),
        pl.BlockSpec((GB, C, D), lambda i: (i, 0, 0)),
        pl.BlockSpec((GB, APG, C), lambda i: (i, 0, 0)),
        pl.BlockSpec((GB, NPG, C), lambda i: (i, 0, 0)),
    )
    return pl.pallas_call(
        _tc_post_body, grid=grid, in_specs=in_specs, out_specs=out_specs,
        out_shape=out_shapes,
    )(nf3, agg4, qs2, W_self, Wn_lo, Wn_hi, b_w, W1, b1, W2, b2)


def kernel(node_feature, qs, edge_index, graph_id, ally_indices,
           W_self, W_nbr, b_w, W1, b1, W2, b2):
    del graph_id, ally_indices

    pad = E_PAD - E
    src3 = jnp.concatenate([edge_index[0], jnp.zeros((pad,), jnp.int32)]
                           ).reshape(NSUB, PER_W, CHUNK)
    dst3 = jnp.concatenate([edge_index[1], jnp.full((pad,), N, jnp.int32)]
                           ).reshape(NSUB, PER_W, CHUNK)
    znf = jnp.stack([node_feature[:, :HD], node_feature[:, HD:]])
    zeros_acc = jnp.zeros((ROWS_PER_SUB, HD), jnp.float32)
    agg_halves = _sc_edge_agg(znf, src3, dst3, zeros_acc)

    nf3 = node_feature.reshape(B, NPG, D)
    agg4 = agg_halves.reshape(2, B, NPG, HD)
    qs2 = qs.reshape(B, APG, 1)
    q_out, ws3, wf, an3, normed3 = _tc_post(
        nf3, agg4, qs2, W_self, W_nbr[:HD], W_nbr[HD:], b_w.reshape(1, C),
        W1, b1.reshape(1, H), W2, b2.reshape(1, 1))

    return (q_out.reshape(B, C), ws3.reshape(B * APG, C), wf,
            an3.reshape(B * APG, C), normed3.reshape(N, C))
